# baseline (speedup 1.0000x reference)
import functools
import numpy as np
import jax
import jax.numpy as jnp
from jax import lax
from jax.experimental import pallas as pl
from jax.experimental.pallas import tpu as pltpu
from jax.experimental.pallas import tpu_sc as plsc

F = 26
D = 64
B = 4096
N = B * F
FD = F * D
G = F * F
NC, NS = 2, 16
NW = NC * NS
N_PER_W = N // NW
CH = 104
NCH = N_PER_W // CH
NG = 4
GC = NCH // NG
GR = GC * CH


def _sc_gather(table, idx2):
  mesh = plsc.VectorSubcoreMesh(core_axis_name="c", subcore_axis_name="s")

  @functools.partial(
      pl.kernel,
      out_type=jax.ShapeDtypeStruct((N, D), jnp.bfloat16),
      mesh=mesh,
      compiler_params=pltpu.CompilerParams(use_tc_tiling_on_sc=False),
      scratch_types=[
          pltpu.VMEM((NCH, CH), jnp.int32),
          pltpu.VMEM((GR, D), jnp.bfloat16),
          pltpu.VMEM((GR, D), jnp.bfloat16),
          pltpu.SemaphoreType.DMA,
          pltpu.SemaphoreType.DMA,
          pltpu.SemaphoreType.DMA,
      ],
  )
  def k(table_hbm, idx_hbm, out_hbm, idx_v, rows_a, rows_b, gsem,
        osem_a, osem_b):
    wid = lax.axis_index("s") * NC + lax.axis_index("c")
    base = wid * N_PER_W
    bufs = (rows_a, rows_b)
    osems = (osem_a, osem_b)
    pltpu.sync_copy(idx_hbm.at[wid], idx_v)
    for g in range(NG):
      buf = bufs[g % 2]
      osem = osems[g % 2]
      if g >= 2:
        pltpu.make_async_copy(
            bufs[g % 2], out_hbm.at[pl.ds(base + (g - 2) * GR, GR)],
            osem).wait()
      for c in range(GC):
        pltpu.async_copy(
            table_hbm.at[idx_v.at[g * GC + c]],
            buf.at[pl.ds(c * CH, CH)],
            gsem,
        )
      for c in range(GC):
        pltpu.make_async_copy(
            table_hbm.at[idx_v.at[g * GC + c]],
            buf.at[pl.ds(c * CH, CH)],
            gsem,
        ).wait()
      pltpu.async_copy(buf, out_hbm.at[pl.ds(base + g * GR, GR)], osem)
    for g in (NG - 2, NG - 1):
      pltpu.make_async_copy(
          bufs[g % 2], out_hbm.at[pl.ds(base + g * GR, GR)],
          osems[g % 2]).wait()

  return k(table, idx2)


def _tc_body(arch_ref, xv_ref, w1a_ref, w1b_ref, b1_ref,
             w2_ref, b2_ref, w3_ref, b3_ref, wo_ref, bo_ref, out_ref):
  R = out_ref.shape[0]
  a = arch_ref[...]
  m = jnp.max(a, axis=1, keepdims=True)
  e = jnp.exp(a - m)
  prob = e / jnp.sum(e, axis=1, keepdims=True)

  ids = lax.broadcasted_iota(jnp.int32, (1, FD), 1) // D
  scale = jnp.zeros((1, FD), jnp.float32)
  for f in range(F):
    pf = lax.slice(prob, (0, f), (1, f + 1))
    scale = scale + jnp.where(ids == f, pf, 0.0)

  xe = (xv_ref[...] * scale).astype(jnp.bfloat16)
  xe3 = xe.reshape(R, F, D)
  gram = lax.dot_general(
      xe3, xe3, (((2,), (2,)), ((0,), (0,))),
      preferred_element_type=jnp.float32)
  gramf = gram.reshape(R, G).astype(jnp.bfloat16)

  h = jnp.maximum(
      jnp.dot(xe, w1a_ref[...], preferred_element_type=jnp.float32)
      + jnp.dot(gramf, w1b_ref[...], preferred_element_type=jnp.float32)
      + b1_ref[...], 0.0).astype(jnp.bfloat16)
  h = jnp.maximum(
      jnp.dot(h, w2_ref[...], preferred_element_type=jnp.float32)
      + b2_ref[...], 0.0).astype(jnp.bfloat16)
  h = jnp.maximum(
      jnp.dot(h, w3_ref[...], preferred_element_type=jnp.float32)
      + b3_ref[...], 0.0)
  out_ref[...] = (
      jnp.sum(h * wo_ref[...], axis=1, keepdims=True) + bo_ref[...])


def kernel(x, beta, arch, embedding, W1, b1, W2, b2, W3, b3, Wo, bo):
  idx2 = x.reshape(NW, NCH, CH).astype(jnp.int32)
  xvflat = _sc_gather(embedding.astype(jnp.bfloat16), idx2)
  xv = xvflat.reshape(B, FD)

  arch_p = jnp.full((1, 128), -1e30, jnp.float32)
  arch_p = arch_p.at[0, :F].set(arch / beta)

  rows, cols = np.triu_indices(F, k=1)
  pos = rows * F + cols
  W1a = W1[:FD].astype(jnp.bfloat16)
  W1b = (jnp.zeros((G, 1024), jnp.float32).at[pos].set(W1[FD:])
         .astype(jnp.bfloat16))

  R = 512
  grid = (B // R,)
  out = pl.pallas_call(
      _tc_body,
      grid=grid,
      in_specs=[
          pl.BlockSpec((1, 128), lambda i: (0, 0)),
          pl.BlockSpec((R, FD), lambda i: (i, 0)),
          pl.BlockSpec((FD, 1024), lambda i: (0, 0)),
          pl.BlockSpec((G, 1024), lambda i: (0, 0)),
          pl.BlockSpec((1, 1024), lambda i: (0, 0)),
          pl.BlockSpec((1024, 512), lambda i: (0, 0)),
          pl.BlockSpec((1, 512), lambda i: (0, 0)),
          pl.BlockSpec((512, 256), lambda i: (0, 0)),
          pl.BlockSpec((1, 256), lambda i: (0, 0)),
          pl.BlockSpec((1, 256), lambda i: (0, 0)),
          pl.BlockSpec((1, 1), lambda i: (0, 0)),
      ],
      out_specs=pl.BlockSpec((R, 1), lambda i: (i, 0)),
      out_shape=jax.ShapeDtypeStruct((B, 1), jnp.float32),
      compiler_params=pltpu.CompilerParams(
          dimension_semantics=("parallel",),
      ),
  )(arch_p, xv, W1a, W1b, b1.reshape(1, 1024),
    W2.astype(jnp.bfloat16), b2.reshape(1, 512),
    W3.astype(jnp.bfloat16), b3.reshape(1, 256), Wo.reshape(1, 256),
    bo.reshape(1, 1))
  return out[:, 0]


# speedup vs baseline: 1.2550x; 1.2550x over previous
import functools
import numpy as np
import jax
import jax.numpy as jnp
from jax import lax
from jax.experimental import pallas as pl
from jax.experimental.pallas import tpu as pltpu
from jax.experimental.pallas import tpu_sc as plsc

F = 26
D = 64
B = 4096
N = B * F
FD = F * D
G = F * F
NC, NS = 2, 16
NW = NC * NS
N_PER_W = N // NW
CH = 128
NCH = N_PER_W // CH
HALF = NCH // 2


def _sc_gather(table, idx2):
  mesh = plsc.VectorSubcoreMesh(core_axis_name="c", subcore_axis_name="s")

  @functools.partial(
      pl.kernel,
      out_type=jax.ShapeDtypeStruct((N, D), jnp.float32),
      mesh=mesh,
      compiler_params=pltpu.CompilerParams(use_tc_tiling_on_sc=False),
      scratch_types=[
          pltpu.VMEM((NCH, CH), jnp.int32),
          pltpu.VMEM((HALF * CH, D), jnp.float32),
          pltpu.SemaphoreType.DMA,
      ],
  )
  def k(table_hbm, idx_hbm, out_hbm, idx_v, rows_v, sem):
    wid = lax.axis_index("s") * NC + lax.axis_index("c")
    base = wid * N_PER_W
    pltpu.sync_copy(idx_hbm.at[wid], idx_v)
    for h in range(2):
      for c in range(HALF):
        j = h * HALF + c
        pltpu.async_copy(
            table_hbm.at[idx_v.at[j]],
            rows_v.at[pl.ds(c * CH, CH)],
            sem,
        )
      for c in range(HALF):
        pltpu.make_async_copy(
            table_hbm.at[idx_v.at[h * HALF + c]],
            rows_v.at[pl.ds(c * CH, CH)],
            sem,
        ).wait()
      pltpu.sync_copy(
          rows_v, out_hbm.at[pl.ds(base + h * HALF * CH, HALF * CH)]
      )

  return k(table, idx2)


def _tc_body(arch_ref, xv_ref, w1a_ref, w1b_ref, b1_ref,
             w2_ref, b2_ref, w3_ref, b3_ref, wo_ref, bo_ref, out_ref):
  R = out_ref.shape[0]
  a = arch_ref[...]
  m = jnp.max(a, axis=1, keepdims=True)
  e = jnp.exp(a - m)
  prob = e / jnp.sum(e, axis=1, keepdims=True)

  ids = lax.broadcasted_iota(jnp.int32, (1, FD), 1) // D
  scale = jnp.zeros((1, FD), jnp.float32)
  for f in range(F):
    pf = lax.slice(prob, (0, f), (1, f + 1))
    scale = scale + jnp.where(ids == f, pf, 0.0)

  xe = (xv_ref[...] * scale).astype(jnp.bfloat16)
  xe3 = xe.reshape(R, F, D)
  gram = lax.dot_general(
      xe3, xe3, (((2,), (2,)), ((0,), (0,))),
      preferred_element_type=jnp.float32)
  gramf = gram.reshape(R, G).astype(jnp.bfloat16)

  h = jnp.maximum(
      jnp.dot(xe, w1a_ref[...], preferred_element_type=jnp.float32)
      + jnp.dot(gramf, w1b_ref[...], preferred_element_type=jnp.float32)
      + b1_ref[...], 0.0).astype(jnp.bfloat16)
  h = jnp.maximum(
      jnp.dot(h, w2_ref[...], preferred_element_type=jnp.float32)
      + b2_ref[...], 0.0).astype(jnp.bfloat16)
  h = jnp.maximum(
      jnp.dot(h, w3_ref[...], preferred_element_type=jnp.float32)
      + b3_ref[...], 0.0)
  out_ref[...] = (
      jnp.sum(h * wo_ref[...], axis=1, keepdims=True) + bo_ref[...])


def kernel(x, beta, arch, embedding, W1, b1, W2, b2, W3, b3, Wo, bo):
  idx2 = x.reshape(NW, NCH, CH).astype(jnp.int32)
  xvflat = _sc_gather(embedding, idx2)
  xv = xvflat.reshape(B, FD)

  arch_p = jnp.full((1, 128), -1e30, jnp.float32)
  arch_p = arch_p.at[0, :F].set(arch / beta)

  rows, cols = np.triu_indices(F, k=1)
  pos = rows * F + cols
  W1a = W1[:FD].astype(jnp.bfloat16)
  W1b = (jnp.zeros((G, 1024), jnp.float32).at[pos].set(W1[FD:])
         .astype(jnp.bfloat16))

  R = 512
  grid = (B // R,)
  out = pl.pallas_call(
      _tc_body,
      grid=grid,
      in_specs=[
          pl.BlockSpec((1, 128), lambda i: (0, 0)),
          pl.BlockSpec((R, FD), lambda i: (i, 0)),
          pl.BlockSpec((FD, 1024), lambda i: (0, 0)),
          pl.BlockSpec((G, 1024), lambda i: (0, 0)),
          pl.BlockSpec((1, 1024), lambda i: (0, 0)),
          pl.BlockSpec((1024, 512), lambda i: (0, 0)),
          pl.BlockSpec((1, 512), lambda i: (0, 0)),
          pl.BlockSpec((512, 256), lambda i: (0, 0)),
          pl.BlockSpec((1, 256), lambda i: (0, 0)),
          pl.BlockSpec((1, 256), lambda i: (0, 0)),
          pl.BlockSpec((1, 1), lambda i: (0, 0)),
      ],
      out_specs=pl.BlockSpec((R, 1), lambda i: (i, 0)),
      out_shape=jax.ShapeDtypeStruct((B, 1), jnp.float32),
      compiler_params=pltpu.CompilerParams(
          dimension_semantics=("parallel",),
      ),
  )(arch_p, xv, W1a, W1b, b1.reshape(1, 1024),
    W2.astype(jnp.bfloat16), b2.reshape(1, 512),
    W3.astype(jnp.bfloat16), b3.reshape(1, 256), Wo.reshape(1, 256),
    bo.reshape(1, 1))
  return out[:, 0]
